# baseline (speedup 1.0000x reference)
import functools

import jax
import jax.numpy as jnp
from jax import lax
from jax.experimental import pallas as pl
from jax.experimental.pallas import tpu as pltpu
from jax.experimental.pallas import tpu_sc as plsc

_NC = 2
_NS = 16
_NW = _NC * _NS
_L = 16

_N = 128 * 1024 * 8
_CHUNK = _N // _NW
_GROUP = 8 * _L
_NGROUPS = _CHUNK // _GROUP

_MAGIC = 8388608.0


def _e8_body(x_hbm, out_hbm, xv, ov):
    wid = lax.axis_index("s") * _NC + lax.axis_index("c")
    base = wid * _CHUNK
    pltpu.sync_copy(x_hbm.at[pl.ds(base, _CHUNK)], xv)

    lane8 = lax.iota(jnp.int32, _L) * 8
    one = 1.0
    half = 0.5
    zero = 0.0

    @plsc.parallel_loop(0, _NGROUPS, 1, unroll=2)
    def step(g):
        goff = g * _GROUP
        idx = [lane8 + (goff + j) for j in range(8)]
        x = [plsc.load_gather(xv, [idx[j]]) for j in range(8)]

        s = [jnp.where(x[j] >= zero, _MAGIC, -_MAGIC) for j in range(8)]
        f0 = [(x[j] + s[j]) - s[j] for j in range(8)]
        d0 = [x[j] - f0[j] for j in range(8)]
        neg = [d0[j] < zero for j in range(8)]
        f1 = [f0[j] - jnp.where(neg[j], one, zero) for j in range(8)]
        d1 = [d0[j] + jnp.where(neg[j], half, -half) for j in range(8)]

        def coset(f, d):
            p = f[0] + f[1]
            for j in range(2, 8):
                p = p + f[j]
            odd = (p.astype(jnp.int32) & 1) == 1
            ad = [jnp.abs(d[j]) for j in range(8)]
            best = ad[0]
            bidx = jnp.zeros((_L,), jnp.int32)
            for j in range(1, 8):
                gt = ad[j] > best
                best = jnp.maximum(best, ad[j])
                bidx = jnp.where(gt, j, bidx)
            g_out = []
            for j in range(8):
                fix = (bidx == j) & odd
                sgn = jnp.where(d[j] >= zero, one, -one)
                g_out.append(f[j] + jnp.where(fix, sgn, zero))
            dist = d[0] * d[0]
            for j in range(1, 8):
                dist = dist + d[j] * d[j]
            dist = dist + jnp.where(odd, one - (best + best), zero)
            return g_out, dist

        g0, dist0 = coset(f0, d0)
        g1, dist1 = coset(f1, d1)
        pick0 = dist0 <= dist1
        for j in range(8):
            out_j = jnp.where(pick0, g0[j], g1[j] + half)
            plsc.store_scatter(ov, [idx[j]], out_j)

    pltpu.sync_copy(ov, out_hbm.at[pl.ds(base, _CHUNK)])


_e8_sc = functools.partial(
    pl.kernel,
    out_type=jax.ShapeDtypeStruct((_N,), jnp.float32),
    mesh=plsc.VectorSubcoreMesh(core_axis_name="c", subcore_axis_name="s"),
    scratch_types=[
        pltpu.VMEM((_CHUNK,), jnp.float32),
        pltpu.VMEM((_CHUNK,), jnp.float32),
    ],
    compiler_params=pltpu.CompilerParams(needs_layout_passes=False),
)(_e8_body)


@jax.jit
def kernel(x):
    if x.shape[-1] != 8:
        raise ValueError(f"E8 expects [..., 8] input, got shape {x.shape}")
    original_shape = x.shape
    y = _e8_sc(x.reshape(-1))
    return y.reshape(original_shape)


# speedup vs baseline: 4.3769x; 4.3769x over previous
import functools

import jax
import jax.numpy as jnp
from jax import lax
from jax.experimental import pallas as pl
from jax.experimental.pallas import tpu as pltpu
from jax.experimental.pallas import tpu_sc as plsc

_NC = 2
_NS = 16
_NW = _NC * _NS
_L = 16

_B, _T, _D = 128, 1024, 8
_ROWS = _B // _NW
_NT = _T // _L

_MAGIC = 8388608.0


def _e8_body(x_hbm, out_hbm, xv, ov):
    wid = lax.axis_index("s") * _NC + lax.axis_index("c")
    row0 = wid * _ROWS
    pltpu.sync_copy(x_hbm.at[pl.ds(row0, _ROWS)], xv)

    one = 1.0
    half = 0.5
    zero = 0.0

    for b in range(_ROWS):
        @plsc.parallel_loop(0, _NT, 1, unroll=2)
        def step(t, b=b):
            t0 = t * _L
            x = [xv[b, j, pl.ds(t0, _L)] for j in range(_D)]

            s = [jnp.where(x[j] >= zero, _MAGIC, -_MAGIC) for j in range(_D)]
            f0 = [(x[j] + s[j]) - s[j] for j in range(_D)]
            d0 = [x[j] - f0[j] for j in range(_D)]
            neg = [d0[j] < zero for j in range(_D)]
            f1 = [f0[j] - jnp.where(neg[j], one, zero) for j in range(_D)]
            d1 = [d0[j] + jnp.where(neg[j], half, -half) for j in range(_D)]

            def coset(f, d):
                p = f[0] + f[1]
                for j in range(2, _D):
                    p = p + f[j]
                odd = (p.astype(jnp.int32) & 1) == 1
                ad = [jnp.abs(d[j]) for j in range(_D)]
                best = ad[0]
                bidx = jnp.zeros((_L,), jnp.int32)
                for j in range(1, _D):
                    gt = ad[j] > best
                    best = jnp.maximum(best, ad[j])
                    bidx = jnp.where(gt, j, bidx)
                g_out = []
                for j in range(_D):
                    fix = (bidx == j) & odd
                    sgn = jnp.where(d[j] >= zero, one, -one)
                    g_out.append(f[j] + jnp.where(fix, sgn, zero))
                dist = d[0] * d[0]
                for j in range(1, _D):
                    dist = dist + d[j] * d[j]
                dist = dist + jnp.where(odd, one - (best + best), zero)
                return g_out, dist

            g0, dist0 = coset(f0, d0)
            g1, dist1 = coset(f1, d1)
            pick0 = dist0 <= dist1
            for j in range(_D):
                ov[b, j, pl.ds(t0, _L)] = jnp.where(pick0, g0[j], g1[j] + half)

    pltpu.sync_copy(ov, out_hbm.at[pl.ds(row0, _ROWS)])


_e8_sc = functools.partial(
    pl.kernel,
    out_type=jax.ShapeDtypeStruct((_B, _D, _T), jnp.float32),
    mesh=plsc.VectorSubcoreMesh(core_axis_name="c", subcore_axis_name="s"),
    scratch_types=[
        pltpu.VMEM((_ROWS, _D, _T), jnp.float32),
        pltpu.VMEM((_ROWS, _D, _T), jnp.float32),
    ],
    compiler_params=pltpu.CompilerParams(needs_layout_passes=False),
)(_e8_body)


@jax.jit
def kernel(x):
    if x.shape[-1] != 8:
        raise ValueError(f"E8 expects [..., 8] input, got shape {x.shape}")
    y_t = _e8_sc(jnp.transpose(x, (0, 2, 1)))
    return jnp.transpose(y_t, (0, 2, 1))
